# Initial kernel scaffold; baseline (speedup 1.0000x reference)
import jax
import jax.numpy as jnp
from jax.experimental import pallas as pl


def kernel(h_prev, observation, kc_ids, problem_ids, student_ids, pT_logit, pF_logit, pG_logit, pS_logit, omega, sigma, student_ability):
    raise NotImplementedError("write your pallas kernel here")


# speedup vs baseline: 5.0730x; 5.0730x over previous
import functools

import jax
import jax.numpy as jnp
from jax import lax
from jax.experimental import pallas as pl
from jax.experimental.pallas import tpu as pltpu
from jax.experimental.pallas import tpu_sc as plsc

B = 16384
NUM_KCS = 1000
KC_PAD = 1024
EPSILON = 1e-08

NC = 2
NS = 16
NW = NC * NS
BPW = B // NW
CHUNK = 128
NCHUNK = BPW // CHUNK
STEPS = BPW // 16


def _sigmoid(x):
    return 1.0 / (1.0 + jnp.exp(-x))


def _bkt_body(kctab_hbm, omega_hbm, sigma_hbm,
              th0_hbm, th1_hbm, th2_hbm, th3_hbm, hp_hbm, obs_hbm,
              kc_hbm, pid2_hbm, sid2_hbm,
              hnew_hbm, pc_hbm,
              kctab_v, kc_v, obs_v, hp_v, pidx_v, sidx_v,
              om_v, sg_v, th0_v, th1_v, th2_v, th3_v, hn_v, pc_v, sem):
    wid = lax.axis_index("s") * NC + lax.axis_index("c")
    base = wid * BPW
    base2 = wid * (2 * BPW)
    row4 = wid * NCHUNK

    copies = [
        pltpu.async_copy(kc_hbm.at[pl.ds(base, BPW)], kc_v, sem),
        pltpu.async_copy(obs_hbm.at[pl.ds(base, BPW)], obs_v, sem),
        pltpu.async_copy(hp_hbm.at[pl.ds(base2, 2 * BPW)], hp_v, sem),
        pltpu.async_copy(kctab_hbm, kctab_v, sem),
        pltpu.async_copy(pid2_hbm.at[pl.ds(row4, NCHUNK)], pidx_v, sem),
        pltpu.async_copy(sid2_hbm.at[pl.ds(row4, NCHUNK)], sidx_v, sem),
    ]
    for c in copies:
        c.wait()

    gathers = []
    for j in range(NCHUNK):
        gathers.append(pltpu.async_copy(
            omega_hbm.at[pidx_v.at[j]], om_v.at[pl.ds(j * CHUNK, CHUNK)], sem))
        gathers.append(pltpu.async_copy(
            sigma_hbm.at[pidx_v.at[j]], sg_v.at[pl.ds(j * CHUNK, CHUNK)], sem))
        gathers.append(pltpu.async_copy(
            th0_hbm.at[sidx_v.at[j]], th0_v.at[pl.ds(j * CHUNK, CHUNK)], sem))
        gathers.append(pltpu.async_copy(
            th1_hbm.at[sidx_v.at[j]], th1_v.at[pl.ds(j * CHUNK, CHUNK)], sem))
        gathers.append(pltpu.async_copy(
            th2_hbm.at[sidx_v.at[j]], th2_v.at[pl.ds(j * CHUNK, CHUNK)], sem))
        gathers.append(pltpu.async_copy(
            th3_hbm.at[sidx_v.at[j]], th3_v.at[pl.ds(j * CHUNK, CHUNK)], sem))
    for g in gathers:
        g.wait()

    zeros16 = jnp.zeros((16,), jnp.int32)
    ones16 = jnp.full((16,), 1, jnp.int32)
    iota16 = lax.iota(jnp.int32, 16)

    def step(i, carry):
        s = pl.multiple_of(i * 16, 16)
        riota = iota16 + i * 16

        kidx = kc_v[pl.ds(s, 16)]
        pT_l = plsc.load_gather(kctab_v, [kidx])
        pF_l = plsc.load_gather(kctab_v, [kidx + KC_PAD])
        pG_l = plsc.load_gather(kctab_v, [kidx + 2 * KC_PAD])
        pS_l = plsc.load_gather(kctab_v, [kidx + 3 * KC_PAD])

        om = om_v[pl.ds(s, 16)]
        sg = sg_v[pl.ds(s, 16)]

        th_L = th0_v[pl.ds(s, 16)]
        th_nF = th1_v[pl.ds(s, 16)]
        th_G = th2_v[pl.ds(s, 16)]
        th_nS = th3_v[pl.ds(s, 16)]

        riota2 = riota * 2
        h_u = plsc.load_gather(hp_v, [riota2])
        h_m = plsc.load_gather(hp_v, [riota2 + 1])

        obs = obs_v[pl.ds(s, 16)]
        obs_b = obs != 0

        pT = _sigmoid(pT_l + th_L)
        pF = _sigmoid(pF_l - th_nF)
        pG = _sigmoid(pG_l + om + th_G)
        pS = _sigmoid(pS_l + sg - th_nS)

        p_m = jnp.where(obs_b, 1.0 - pS, pS)
        p_u = jnp.where(obs_b, pG, 1.0 - pG)

        a_u = p_u * h_u
        a_m = p_m * h_m
        new_m = (1.0 - pF) * a_m + pT * a_u
        new_u = pF * a_m + (1.0 - pT) * a_u
        inv_norm = 1.0 / (new_m + new_u + EPSILON)
        new_m = new_m * inv_norm
        new_u = new_u * inv_norm
        pc = (1.0 - pS) * new_m + pG * new_u

        plsc.store_scatter(hn_v, [riota2], new_u)
        plsc.store_scatter(hn_v, [riota2 + 1], new_m)
        pc_v[pl.ds(s, 16)] = pc
        return carry

    lax.fori_loop(0, STEPS, step, 0)

    pltpu.async_copy(hn_v, hnew_hbm.at[pl.ds(base2, 2 * BPW)], sem).wait()
    pltpu.async_copy(pc_v, pc_hbm.at[pl.ds(base, BPW)], sem).wait()


@jax.jit
def _bkt_sc(kctab, omega1d, sigma1d, th0, th1, th2, th3, h_prev, observation,
            kc_ids, pid2, sid2):
    mesh = plsc.VectorSubcoreMesh(core_axis_name="c", subcore_axis_name="s",
                                  num_cores=NC, num_subcores=NS)
    fn = pl.kernel(
        _bkt_body,
        out_type=(
            jax.ShapeDtypeStruct((2 * B,), jnp.float32),
            jax.ShapeDtypeStruct((B,), jnp.float32),
        ),
        mesh=mesh,
        scratch_types=[
            pltpu.VMEM((4 * KC_PAD,), jnp.float32),
            pltpu.VMEM((BPW,), jnp.int32),
            pltpu.VMEM((BPW,), jnp.int32),
            pltpu.VMEM((2 * BPW,), jnp.float32),
            pltpu.VMEM((NCHUNK, CHUNK), jnp.int32),
            pltpu.VMEM((NCHUNK, CHUNK), jnp.int32),
            pltpu.VMEM((BPW,), jnp.float32),
            pltpu.VMEM((BPW,), jnp.float32),
            pltpu.VMEM((BPW,), jnp.float32),
            pltpu.VMEM((BPW,), jnp.float32),
            pltpu.VMEM((BPW,), jnp.float32),
            pltpu.VMEM((BPW,), jnp.float32),
            pltpu.VMEM((2 * BPW,), jnp.float32),
            pltpu.VMEM((BPW,), jnp.float32),
            pltpu.SemaphoreType.DMA,
        ],
        compiler_params=pltpu.CompilerParams(needs_layout_passes=False),
        name="bkt_sc",
    )
    return fn(kctab, omega1d, sigma1d, th0, th1, th2, th3, h_prev,
              observation, kc_ids, pid2, sid2)


def kernel(h_prev, observation, kc_ids, problem_ids, student_ids,
           pT_logit, pF_logit, pG_logit, pS_logit, omega, sigma,
           student_ability):
    pad = KC_PAD - NUM_KCS
    kctab = jnp.concatenate([
        jnp.pad(pT_logit, (0, pad)),
        jnp.pad(pF_logit, (0, pad)),
        jnp.pad(pG_logit, (0, pad)),
        jnp.pad(pS_logit, (0, pad)),
    ])
    omega1d = omega.reshape(-1)
    sigma1d = sigma.reshape(-1)
    pid2 = problem_ids.reshape(B // CHUNK, CHUNK)
    sid2 = student_ids.reshape(B // CHUNK, CHUNK)
    thT = student_ability.T
    h_flat, p_correct = _bkt_sc(kctab, omega1d, sigma1d,
                                thT[0], thT[1], thT[2], thT[3],
                                h_prev.reshape(-1), observation,
                                kc_ids, pid2, sid2)
    return (h_flat.reshape(B, 2), p_correct)


# speedup vs baseline: 5.0819x; 1.0018x over previous
import functools

import jax
import jax.numpy as jnp
from jax import lax
from jax.experimental import pallas as pl
from jax.experimental.pallas import tpu as pltpu
from jax.experimental.pallas import tpu_sc as plsc

B = 16384
NUM_KCS = 1000
KC_PAD = 1024
EPSILON = 1e-08

NC = 2
NS = 16
NW = NC * NS
BPW = B // NW
CHUNK = 128
NCHUNK = BPW // CHUNK
STEPS = BPW // 16


def _sigmoid(x):
    return 1.0 / (1.0 + jnp.exp(-x))


def _bkt_body(kctab_hbm, omega_hbm, sigma_hbm,
              th0_hbm, th1_hbm, th2_hbm, th3_hbm, hp_hbm, obs_hbm,
              kc_hbm, pid2_hbm, sid2_hbm,
              hnew_hbm, pc_hbm,
              kctab_v, kc_v, obs_v, hp_v, pidx_v, sidx_v,
              om_v, sg_v, th0_v, th1_v, th2_v, th3_v, hn_v, pc_v, sem):
    wid = lax.axis_index("s") * NC + lax.axis_index("c")
    base = wid * BPW
    base2 = wid * (2 * BPW)
    row4 = wid * NCHUNK

    copies = [
        pltpu.async_copy(kc_hbm.at[pl.ds(base, BPW)], kc_v, sem),
        pltpu.async_copy(obs_hbm.at[pl.ds(base, BPW)], obs_v, sem),
        pltpu.async_copy(hp_hbm.at[pl.ds(base2, 2 * BPW)], hp_v, sem),
        pltpu.async_copy(kctab_hbm, kctab_v, sem),
        pltpu.async_copy(pid2_hbm.at[pl.ds(row4, NCHUNK)], pidx_v, sem),
        pltpu.async_copy(sid2_hbm.at[pl.ds(row4, NCHUNK)], sidx_v, sem),
    ]
    for c in copies:
        c.wait()

    gathers = []
    for j in range(NCHUNK):
        gathers.append(pltpu.async_copy(
            omega_hbm.at[pidx_v.at[j]], om_v.at[pl.ds(j * CHUNK, CHUNK)], sem))
        gathers.append(pltpu.async_copy(
            sigma_hbm.at[pidx_v.at[j]], sg_v.at[pl.ds(j * CHUNK, CHUNK)], sem))
        gathers.append(pltpu.async_copy(
            th0_hbm.at[sidx_v.at[j]], th0_v.at[pl.ds(j * CHUNK, CHUNK)], sem))
        gathers.append(pltpu.async_copy(
            th1_hbm.at[sidx_v.at[j]], th1_v.at[pl.ds(j * CHUNK, CHUNK)], sem))
        gathers.append(pltpu.async_copy(
            th2_hbm.at[sidx_v.at[j]], th2_v.at[pl.ds(j * CHUNK, CHUNK)], sem))
        gathers.append(pltpu.async_copy(
            th3_hbm.at[sidx_v.at[j]], th3_v.at[pl.ds(j * CHUNK, CHUNK)], sem))
    for g in gathers:
        g.wait()

    zeros16 = jnp.zeros((16,), jnp.int32)
    ones16 = jnp.full((16,), 1, jnp.int32)
    iota16 = lax.iota(jnp.int32, 16)

    def step(i, carry):
        s = pl.multiple_of(i * 16, 16)
        riota = iota16 + i * 16

        kidx = kc_v[pl.ds(s, 16)]
        pT_l = plsc.load_gather(kctab_v, [kidx])
        pF_l = plsc.load_gather(kctab_v, [kidx + KC_PAD])
        pG_l = plsc.load_gather(kctab_v, [kidx + 2 * KC_PAD])
        pS_l = plsc.load_gather(kctab_v, [kidx + 3 * KC_PAD])

        om = om_v[pl.ds(s, 16)]
        sg = sg_v[pl.ds(s, 16)]

        th_L = th0_v[pl.ds(s, 16)]
        th_nF = th1_v[pl.ds(s, 16)]
        th_G = th2_v[pl.ds(s, 16)]
        th_nS = th3_v[pl.ds(s, 16)]

        riota2 = riota * 2
        h_u = plsc.load_gather(hp_v, [riota2])
        h_m = plsc.load_gather(hp_v, [riota2 + 1])

        obs = obs_v[pl.ds(s, 16)]
        obs_b = obs != 0

        pT = _sigmoid(pT_l + th_L)
        pF = _sigmoid(pF_l - th_nF)
        pG = _sigmoid(pG_l + om + th_G)
        pS = _sigmoid(pS_l + sg - th_nS)

        p_m = jnp.where(obs_b, 1.0 - pS, pS)
        p_u = jnp.where(obs_b, pG, 1.0 - pG)

        a_u = p_u * h_u
        a_m = p_m * h_m
        new_m = (1.0 - pF) * a_m + pT * a_u
        new_u = pF * a_m + (1.0 - pT) * a_u
        inv_norm = 1.0 / (new_m + new_u + EPSILON)
        new_m = new_m * inv_norm
        new_u = new_u * inv_norm
        pc = (1.0 - pS) * new_m + pG * new_u

        plsc.store_scatter(hn_v, [riota2], new_u)
        plsc.store_scatter(hn_v, [riota2 + 1], new_m)
        pc_v[pl.ds(s, 16)] = pc
        return carry

    lax.fori_loop(0, STEPS, step, 0)

    pltpu.async_copy(hn_v, hnew_hbm.at[pl.ds(base2, 2 * BPW)], sem).wait()
    pltpu.async_copy(pc_v, pc_hbm.at[pl.ds(base, BPW)], sem).wait()


@jax.jit
def _bkt_sc(kctab, omega1d, sigma1d, th0, th1, th2, th3, h_prev, observation,
            kc_ids, pid2, sid2):
    mesh = plsc.VectorSubcoreMesh(core_axis_name="c", subcore_axis_name="s",
                                  num_cores=NC, num_subcores=NS)
    fn = pl.kernel(
        _bkt_body,
        out_type=(
            jax.ShapeDtypeStruct((2 * B,), jnp.float32),
            jax.ShapeDtypeStruct((B,), jnp.float32),
        ),
        mesh=mesh,
        scratch_types=[
            pltpu.VMEM((4 * KC_PAD,), jnp.float32),
            pltpu.VMEM((BPW,), jnp.int32),
            pltpu.VMEM((BPW,), jnp.int32),
            pltpu.VMEM((2 * BPW,), jnp.float32),
            pltpu.VMEM((NCHUNK, CHUNK), jnp.int32),
            pltpu.VMEM((NCHUNK, CHUNK), jnp.int32),
            pltpu.VMEM((BPW,), jnp.float32),
            pltpu.VMEM((BPW,), jnp.float32),
            pltpu.VMEM((BPW,), jnp.float32),
            pltpu.VMEM((BPW,), jnp.float32),
            pltpu.VMEM((BPW,), jnp.float32),
            pltpu.VMEM((BPW,), jnp.float32),
            pltpu.VMEM((2 * BPW,), jnp.float32),
            pltpu.VMEM((BPW,), jnp.float32),
            pltpu.SemaphoreType.DMA,
        ],
        compiler_params=pltpu.CompilerParams(needs_layout_passes=False,
                                             disable_bounds_checks=True),
        name="bkt_sc",
    )
    return fn(kctab, omega1d, sigma1d, th0, th1, th2, th3, h_prev,
              observation, kc_ids, pid2, sid2)


def kernel(h_prev, observation, kc_ids, problem_ids, student_ids,
           pT_logit, pF_logit, pG_logit, pS_logit, omega, sigma,
           student_ability):
    pad = KC_PAD - NUM_KCS
    kctab = jnp.concatenate([
        jnp.pad(pT_logit, (0, pad)),
        jnp.pad(pF_logit, (0, pad)),
        jnp.pad(pG_logit, (0, pad)),
        jnp.pad(pS_logit, (0, pad)),
    ])
    omega1d = omega.reshape(-1)
    sigma1d = sigma.reshape(-1)
    pid2 = problem_ids.reshape(B // CHUNK, CHUNK)
    sid2 = student_ids.reshape(B // CHUNK, CHUNK)
    thT = student_ability.T
    h_flat, p_correct = _bkt_sc(kctab, omega1d, sigma1d,
                                thT[0], thT[1], thT[2], thT[3],
                                h_prev.reshape(-1), observation,
                                kc_ids, pid2, sid2)
    return (h_flat.reshape(B, 2), p_correct)


# speedup vs baseline: 6.2965x; 1.2390x over previous
import jax
import jax.numpy as jnp
from jax import lax
from jax.experimental import pallas as pl
from jax.experimental.pallas import tpu as pltpu
from jax.experimental.pallas import tpu_sc as plsc

B = 16384
NUM_KCS = 1000
KC_PAD = 1024
EPSILON = 1e-08

NC = 2
NS = 16
NW = NC * NS
BPW = B // NW
CHUNK = 128
NCHUNK = BPW // CHUNK
STEPS = BPW // 16


def _sigmoid(x):
    return 1.0 / (1.0 + jnp.exp(-x))


def _bkt_body(kctab_hbm, omega_hbm, sigma_hbm,
              th0_hbm, th1_hbm, th2_hbm, th3_hbm, hp_hbm, obs_hbm,
              kc_hbm, pid_hbm, sid_hbm,
              hnew_hbm, pc_hbm,
              kctab_v, kc_v, obs_v, hp_v, pidx_v, sidx_v,
              om_v, sg_v, th0_v, th1_v, th2_v, th3_v, hn_v, pc_v, sem):
    wid = lax.axis_index("s") * NC + lax.axis_index("c")
    base = wid * BPW
    base2 = wid * (2 * BPW)

    copies = [
        pltpu.async_copy(kc_hbm.at[pl.ds(base, BPW)], kc_v, sem),
        pltpu.async_copy(obs_hbm.at[pl.ds(base, BPW)], obs_v, sem),
        pltpu.async_copy(hp_hbm.at[pl.ds(base2, 2 * BPW)], hp_v, sem),
        pltpu.async_copy(kctab_hbm, kctab_v, sem),
        pltpu.async_copy(pid_hbm.at[pl.ds(base, BPW)], pidx_v, sem),
        pltpu.async_copy(sid_hbm.at[pl.ds(base, BPW)], sidx_v, sem),
    ]
    for c in copies:
        c.wait()

    gathers = []
    for j in range(NCHUNK):
        sl = pl.ds(j * CHUNK, CHUNK)
        gathers.append(pltpu.async_copy(
            omega_hbm.at[pidx_v.at[sl]], om_v.at[sl], sem))
        gathers.append(pltpu.async_copy(
            sigma_hbm.at[pidx_v.at[sl]], sg_v.at[sl], sem))
        gathers.append(pltpu.async_copy(
            th0_hbm.at[sidx_v.at[sl]], th0_v.at[sl], sem))
        gathers.append(pltpu.async_copy(
            th1_hbm.at[sidx_v.at[sl]], th1_v.at[sl], sem))
        gathers.append(pltpu.async_copy(
            th2_hbm.at[sidx_v.at[sl]], th2_v.at[sl], sem))
        gathers.append(pltpu.async_copy(
            th3_hbm.at[sidx_v.at[sl]], th3_v.at[sl], sem))
    for g in gathers:
        g.wait()

    def step(i, carry):
        s = pl.multiple_of(i * 16, 16)
        su = pl.multiple_of((i // 8) * 256 + (i % 8) * 16, 16)

        kidx = kc_v[pl.ds(s, 16)]
        pT_l = plsc.load_gather(kctab_v, [kidx])
        pF_l = plsc.load_gather(kctab_v, [kidx + KC_PAD])
        pG_l = plsc.load_gather(kctab_v, [kidx + 2 * KC_PAD])
        pS_l = plsc.load_gather(kctab_v, [kidx + 3 * KC_PAD])

        om = om_v[pl.ds(s, 16)]
        sg = sg_v[pl.ds(s, 16)]

        th_L = th0_v[pl.ds(s, 16)]
        th_nF = th1_v[pl.ds(s, 16)]
        th_G = th2_v[pl.ds(s, 16)]
        th_nS = th3_v[pl.ds(s, 16)]

        h_u = hp_v[pl.ds(su, 16)]
        h_m = hp_v[pl.ds(su + CHUNK, 16)]

        obs = obs_v[pl.ds(s, 16)]
        obs_b = obs != 0

        pT = _sigmoid(pT_l + th_L)
        pF = _sigmoid(pF_l - th_nF)
        pG = _sigmoid(pG_l + om + th_G)
        pS = _sigmoid(pS_l + sg - th_nS)

        p_m = jnp.where(obs_b, 1.0 - pS, pS)
        p_u = jnp.where(obs_b, pG, 1.0 - pG)

        a_u = p_u * h_u
        a_m = p_m * h_m
        new_m = (1.0 - pF) * a_m + pT * a_u
        new_u = pF * a_m + (1.0 - pT) * a_u
        inv_norm = 1.0 / (new_m + new_u + EPSILON)
        new_m = new_m * inv_norm
        new_u = new_u * inv_norm
        pc = (1.0 - pS) * new_m + pG * new_u

        hn_v[pl.ds(su, 16)] = new_u
        hn_v[pl.ds(su + CHUNK, 16)] = new_m
        pc_v[pl.ds(s, 16)] = pc
        return carry

    lax.fori_loop(0, STEPS, step, 0)

    pltpu.async_copy(hn_v, hnew_hbm.at[pl.ds(base2, 2 * BPW)], sem).wait()
    pltpu.async_copy(pc_v, pc_hbm.at[pl.ds(base, BPW)], sem).wait()


@jax.jit
def _bkt_sc(kctab, omega1d, sigma1d, th0, th1, th2, th3, hp_blk, observation,
            kc_ids, problem_ids, student_ids):
    mesh = plsc.VectorSubcoreMesh(core_axis_name="c", subcore_axis_name="s",
                                  num_cores=NC, num_subcores=NS)
    fn = pl.kernel(
        _bkt_body,
        out_type=(
            jax.ShapeDtypeStruct((2 * B,), jnp.float32),
            jax.ShapeDtypeStruct((B,), jnp.float32),
        ),
        mesh=mesh,
        scratch_types=[
            pltpu.VMEM((4 * KC_PAD,), jnp.float32),
            pltpu.VMEM((BPW,), jnp.int32),
            pltpu.VMEM((BPW,), jnp.int32),
            pltpu.VMEM((2 * BPW,), jnp.float32),
            pltpu.VMEM((BPW,), jnp.int32),
            pltpu.VMEM((BPW,), jnp.int32),
            pltpu.VMEM((BPW,), jnp.float32),
            pltpu.VMEM((BPW,), jnp.float32),
            pltpu.VMEM((BPW,), jnp.float32),
            pltpu.VMEM((BPW,), jnp.float32),
            pltpu.VMEM((BPW,), jnp.float32),
            pltpu.VMEM((BPW,), jnp.float32),
            pltpu.VMEM((2 * BPW,), jnp.float32),
            pltpu.VMEM((BPW,), jnp.float32),
            pltpu.SemaphoreType.DMA,
        ],
        compiler_params=pltpu.CompilerParams(needs_layout_passes=False,
                                             disable_bounds_checks=True),
        name="bkt_sc",
    )
    return fn(kctab, omega1d, sigma1d, th0, th1, th2, th3, hp_blk,
              observation, kc_ids, problem_ids, student_ids)


def kernel(h_prev, observation, kc_ids, problem_ids, student_ids,
           pT_logit, pF_logit, pG_logit, pS_logit, omega, sigma,
           student_ability):
    pad = KC_PAD - NUM_KCS
    kctab = jnp.concatenate([
        jnp.pad(pT_logit, (0, pad)),
        jnp.pad(pF_logit, (0, pad)),
        jnp.pad(pG_logit, (0, pad)),
        jnp.pad(pS_logit, (0, pad)),
    ])
    omega1d = omega.reshape(-1)
    sigma1d = sigma.reshape(-1)
    thT = student_ability.T
    hp_blk = h_prev.reshape(B // CHUNK, CHUNK, 2).transpose(0, 2, 1).reshape(-1)
    h_blk, p_correct = _bkt_sc(kctab, omega1d, sigma1d,
                               thT[0], thT[1], thT[2], thT[3],
                               hp_blk, observation,
                               kc_ids, problem_ids, student_ids)
    h_new = h_blk.reshape(B // CHUNK, 2, CHUNK).transpose(0, 2, 1).reshape(B, 2)
    return (h_new, p_correct)


# speedup vs baseline: 6.3239x; 1.0044x over previous
import jax
import jax.numpy as jnp
from jax import lax
from jax.experimental import pallas as pl
from jax.experimental.pallas import tpu as pltpu
from jax.experimental.pallas import tpu_sc as plsc

B = 16384
NUM_KCS = 1000
KC_PAD = 1024
EPSILON = 1e-08

NC = 2
NS = 16
NW = NC * NS
BPW = B // NW
CHUNK = 128
NCHUNK = BPW // CHUNK
STEPS = BPW // 16


def _sigmoid(x):
    return 1.0 / (1.0 + jnp.exp(-x))


def _bkt_body(kctab_hbm, omega_hbm, sigma_hbm,
              th0_hbm, th1_hbm, th2_hbm, th3_hbm, hp_hbm, obs_hbm,
              kc_hbm, pid_hbm, sid_hbm,
              hnew_hbm, pc_hbm,
              kctab_v, kc_v, obs_v, hp_v, pidx_v, sidx_v,
              om_v, sg_v, th0_v, th1_v, th2_v, th3_v, hn_v, pc_v,
              sem, gsem):
    wid = lax.axis_index("s") * NC + lax.axis_index("c")
    base = wid * BPW
    base2 = wid * (2 * BPW)

    idx_copies = [
        pltpu.async_copy(pid_hbm.at[pl.ds(base, BPW)], pidx_v, sem),
        pltpu.async_copy(sid_hbm.at[pl.ds(base, BPW)], sidx_v, sem),
    ]
    lin_copies = [
        pltpu.async_copy(kc_hbm.at[pl.ds(base, BPW)], kc_v, sem),
        pltpu.async_copy(obs_hbm.at[pl.ds(base, BPW)], obs_v, sem),
        pltpu.async_copy(hp_hbm.at[pl.ds(base2, 2 * BPW)], hp_v, sem),
        pltpu.async_copy(kctab_hbm, kctab_v, sem),
    ]
    for c in idx_copies:
        c.wait()

    gathers = []
    for j in range(NCHUNK):
        sl = pl.ds(j * CHUNK, CHUNK)
        gathers.append([
            pltpu.async_copy(omega_hbm.at[pidx_v.at[sl]], om_v.at[sl],
                             gsem.at[j]),
            pltpu.async_copy(sigma_hbm.at[pidx_v.at[sl]], sg_v.at[sl],
                             gsem.at[j]),
            pltpu.async_copy(th0_hbm.at[sidx_v.at[sl]], th0_v.at[sl],
                             gsem.at[j]),
            pltpu.async_copy(th1_hbm.at[sidx_v.at[sl]], th1_v.at[sl],
                             gsem.at[j]),
            pltpu.async_copy(th2_hbm.at[sidx_v.at[sl]], th2_v.at[sl],
                             gsem.at[j]),
            pltpu.async_copy(th3_hbm.at[sidx_v.at[sl]], th3_v.at[sl],
                             gsem.at[j]),
        ])
    for c in lin_copies:
        c.wait()

    def step(i, carry):
        s = pl.multiple_of(i * 16, 16)
        su = pl.multiple_of((i // 8) * 256 + (i % 8) * 16, 16)

        kidx = kc_v[pl.ds(s, 16)]
        pT_l = plsc.load_gather(kctab_v, [kidx])
        pF_l = plsc.load_gather(kctab_v, [kidx + KC_PAD])
        pG_l = plsc.load_gather(kctab_v, [kidx + 2 * KC_PAD])
        pS_l = plsc.load_gather(kctab_v, [kidx + 3 * KC_PAD])

        om = om_v[pl.ds(s, 16)]
        sg = sg_v[pl.ds(s, 16)]

        th_L = th0_v[pl.ds(s, 16)]
        th_nF = th1_v[pl.ds(s, 16)]
        th_G = th2_v[pl.ds(s, 16)]
        th_nS = th3_v[pl.ds(s, 16)]

        h_u = hp_v[pl.ds(su, 16)]
        h_m = hp_v[pl.ds(su + CHUNK, 16)]

        obs = obs_v[pl.ds(s, 16)]
        obs_b = obs != 0

        pT = _sigmoid(pT_l + th_L)
        pF = _sigmoid(pF_l - th_nF)
        pG = _sigmoid(pG_l + om + th_G)
        pS = _sigmoid(pS_l + sg - th_nS)

        p_m = jnp.where(obs_b, 1.0 - pS, pS)
        p_u = jnp.where(obs_b, pG, 1.0 - pG)

        a_u = p_u * h_u
        a_m = p_m * h_m
        new_m = (1.0 - pF) * a_m + pT * a_u
        new_u = pF * a_m + (1.0 - pT) * a_u
        inv_norm = 1.0 / (new_m + new_u + EPSILON)
        new_m = new_m * inv_norm
        new_u = new_u * inv_norm
        pc = (1.0 - pS) * new_m + pG * new_u

        hn_v[pl.ds(su, 16)] = new_u
        hn_v[pl.ds(su + CHUNK, 16)] = new_m
        pc_v[pl.ds(s, 16)] = pc
        return carry

    for j in range(NCHUNK):
        for g in gathers[j]:
            g.wait()
        lax.fori_loop(j * (STEPS // NCHUNK), (j + 1) * (STEPS // NCHUNK),
                      step, 0)

    pltpu.async_copy(hn_v, hnew_hbm.at[pl.ds(base2, 2 * BPW)], sem).wait()
    pltpu.async_copy(pc_v, pc_hbm.at[pl.ds(base, BPW)], sem).wait()


@jax.jit
def _bkt_sc(kctab, omega1d, sigma1d, th0, th1, th2, th3, hp_blk, observation,
            kc_ids, problem_ids, student_ids):
    mesh = plsc.VectorSubcoreMesh(core_axis_name="c", subcore_axis_name="s",
                                  num_cores=NC, num_subcores=NS)
    fn = pl.kernel(
        _bkt_body,
        out_type=(
            jax.ShapeDtypeStruct((2 * B,), jnp.float32),
            jax.ShapeDtypeStruct((B,), jnp.float32),
        ),
        mesh=mesh,
        scratch_types=[
            pltpu.VMEM((4 * KC_PAD,), jnp.float32),
            pltpu.VMEM((BPW,), jnp.int32),
            pltpu.VMEM((BPW,), jnp.int32),
            pltpu.VMEM((2 * BPW,), jnp.float32),
            pltpu.VMEM((BPW,), jnp.int32),
            pltpu.VMEM((BPW,), jnp.int32),
            pltpu.VMEM((BPW,), jnp.float32),
            pltpu.VMEM((BPW,), jnp.float32),
            pltpu.VMEM((BPW,), jnp.float32),
            pltpu.VMEM((BPW,), jnp.float32),
            pltpu.VMEM((BPW,), jnp.float32),
            pltpu.VMEM((BPW,), jnp.float32),
            pltpu.VMEM((2 * BPW,), jnp.float32),
            pltpu.VMEM((BPW,), jnp.float32),
            pltpu.SemaphoreType.DMA,
            pltpu.SemaphoreType.DMA((NCHUNK,)),
        ],
        compiler_params=pltpu.CompilerParams(needs_layout_passes=False,
                                             disable_bounds_checks=True),
        name="bkt_sc",
    )
    return fn(kctab, omega1d, sigma1d, th0, th1, th2, th3, hp_blk,
              observation, kc_ids, problem_ids, student_ids)


def kernel(h_prev, observation, kc_ids, problem_ids, student_ids,
           pT_logit, pF_logit, pG_logit, pS_logit, omega, sigma,
           student_ability):
    pad = KC_PAD - NUM_KCS
    kctab = jnp.concatenate([
        jnp.pad(pT_logit, (0, pad)),
        jnp.pad(pF_logit, (0, pad)),
        jnp.pad(pG_logit, (0, pad)),
        jnp.pad(pS_logit, (0, pad)),
    ])
    omega1d = omega.reshape(-1)
    sigma1d = sigma.reshape(-1)
    thT = student_ability.T
    hp_blk = h_prev.reshape(B // CHUNK, CHUNK, 2).transpose(0, 2, 1).reshape(-1)
    h_blk, p_correct = _bkt_sc(kctab, omega1d, sigma1d,
                               thT[0], thT[1], thT[2], thT[3],
                               hp_blk, observation,
                               kc_ids, problem_ids, student_ids)
    h_new = h_blk.reshape(B // CHUNK, 2, CHUNK).transpose(0, 2, 1).reshape(B, 2)
    return (h_new, p_correct)
